# baseline (device time: 69001 ns/iter reference)
import jax
import jax.numpy as jnp
from jax import lax
from jax.experimental import pallas as pl
from jax.experimental.pallas import tpu as pltpu

N_DEV = 8
SQ = 1024
DH = 128
HQ_PER = 8
D_MODEL = 1024
WINDOW = 128
KBAND = 3 * 128
SCALE = 0.08838834764831843
CHUNK = SQ // N_DEV

PART_ROWS = (384, 384, 256)
PART_OFF = (0, 384, 768)
PART_MASKS = ((4, 3, 1), (3, 1, 4), (1, 4, 3))
N_ROUNDS = 3
MAX_HALF = 192


def kernel(x, Wq, K_ext, V_ext, Wo):
    x2 = x[0]

    def body(x_ref, wq_ref, k_any, v_any, wo_ref, out_ref,
             acc_ref, ctx_ref, k_ref, v_ref, rs_buf,
             kv_sems, send_sems, recv_sems):
        my = lax.axis_index("i")

        def side(mask):
            if mask == 4:
                return (my // 4) % 2
            if mask == 3:
                return (my // 2) % 2
            return (my + my // 2) % 2

        kv_dmas = []
        for h in range(HQ_PER):
            hh = my * HQ_PER + h
            for src, dst, sem in ((k_any, k_ref, kv_sems.at[0, h]),
                                  (v_any, v_ref, kv_sems.at[1, h])):
                dma = pltpu.make_async_copy(src.at[0, :, hh, :],
                                            dst.at[h], sem)
                dma.start()
                kv_dmas.append(dma)

        barrier = pltpu.get_barrier_semaphore()
        for mask in (1, 3, 4):
            pl.semaphore_signal(barrier, inc=1, device_id=(my ^ mask,),
                                device_id_type=pl.DeviceIdType.MESH)
        pl.semaphore_wait(barrier, 3)

        q = jnp.dot(x_ref[...], wq_ref[...],
                    preferred_element_type=jnp.float32) * SCALE

        for dma in kv_dmas:
            dma.wait()

        masks = []
        for rel in (0, 128, 256):
            qi = rel + lax.broadcasted_iota(jnp.int32, (CHUNK, KBAND), 0)
            ki = lax.broadcasted_iota(jnp.int32, (CHUNK, KBAND), 1)
            masks.append(jnp.abs(qi - ki) <= WINDOW)
        for cc in range(N_DEV):
            row0 = cc * CHUNK
            start = min(max(row0 - WINDOW, 0), SQ - KBAND)
            band = masks[(row0 - start) // CHUNK]
            for h in range(HQ_PER):
                s = lax.dot_general(q[row0:row0 + CHUNK, h * DH:(h + 1) * DH],
                                    k_ref[h, start:start + KBAND, :],
                                    (((1,), (1,)), ((), ())),
                                    preferred_element_type=jnp.float32)
                w = jnp.exp(jnp.where(band, s, -1e9))
                ctx = jnp.dot(w, v_ref[h, start:start + KBAND, :],
                              preferred_element_type=jnp.float32)
                ctx_ref[pl.ds(row0, CHUNK), h * DH:(h + 1) * DH] = (
                    ctx / jnp.sum(w, axis=1, keepdims=True))

        acc_ref[...] = jnp.dot(ctx_ref[...], wo_ref[...],
                               preferred_element_type=jnp.float32)

        offs = [jnp.int32(0)] * 3
        sizes = list(PART_ROWS)
        pending = []
        for r in range(N_ROUNDS):
            started = []
            for p in range(3):
                mask = PART_MASKS[p][r]
                half = sizes[p] // 2
                b = side(mask)
                send_off = PART_OFF[p] + offs[p] + (1 - b) * half
                keep_off = PART_OFF[p] + offs[p] + b * half
                rdma = pltpu.make_async_remote_copy(
                    src_ref=acc_ref.at[pl.ds(send_off, half), :],
                    dst_ref=rs_buf.at[p, r, pl.ds(0, half), :],
                    send_sem=send_sems.at[p * 6 + r],
                    recv_sem=recv_sems.at[p * 6 + r],
                    device_id=(my ^ mask,),
                    device_id_type=pl.DeviceIdType.MESH,
                )
                rdma.start()
                started.append((rdma, keep_off, half))
                offs[p] = offs[p] + b * half
                sizes[p] = half
            for p, (rdma, keep_off, half) in enumerate(started):
                rdma.wait_recv()
                sl = pl.ds(keep_off, half)
                acc_ref[sl, :] = acc_ref[sl, :] + rs_buf[p, r, :half, :]
                pending.append(rdma)

        for p in range(3):
            sl = pl.ds(PART_OFF[p] + offs[p], sizes[p])
            out_ref[0, sl, :] = acc_ref[sl, :]

        for j in range(N_ROUNDS):
            started = []
            for p in range(3):
                mask = PART_MASKS[p][N_ROUNDS - 1 - j]
                b = side(mask)
                cur = sizes[p]
                sl = pl.ds(PART_OFF[p] + offs[p], cur)
                rdma = pltpu.make_async_remote_copy(
                    src_ref=out_ref.at[0, sl, :],
                    dst_ref=out_ref.at[0, sl, :],
                    send_sem=send_sems.at[p * 6 + N_ROUNDS + j],
                    recv_sem=recv_sems.at[p * 6 + N_ROUNDS + j],
                    device_id=(my ^ mask,),
                    device_id_type=pl.DeviceIdType.MESH,
                )
                rdma.start()
                started.append(rdma)
                offs[p] = offs[p] - b * cur
                sizes[p] = 2 * cur
            for rdma in started:
                rdma.wait_recv()
                pending.append(rdma)

        for rdma in pending:
            rdma.wait_send()

    return pl.pallas_call(
        body,
        out_shape=jax.ShapeDtypeStruct((1, SQ, D_MODEL), jnp.float32),
        in_specs=[
            pl.BlockSpec(memory_space=pltpu.VMEM),
            pl.BlockSpec(memory_space=pltpu.VMEM),
            pl.BlockSpec(memory_space=pltpu.MemorySpace.HBM),
            pl.BlockSpec(memory_space=pltpu.MemorySpace.HBM),
            pl.BlockSpec(memory_space=pltpu.VMEM),
        ],
        out_specs=pl.BlockSpec(memory_space=pltpu.VMEM),
        scratch_shapes=[
            pltpu.VMEM((SQ, D_MODEL), jnp.float32),
            pltpu.VMEM((SQ, D_MODEL), jnp.float32),
            pltpu.VMEM((HQ_PER, SQ, DH), jnp.float32),
            pltpu.VMEM((HQ_PER, SQ, DH), jnp.float32),
            pltpu.VMEM((3, N_ROUNDS, MAX_HALF, D_MODEL), jnp.float32),
            pltpu.SemaphoreType.DMA((2, HQ_PER)),
            pltpu.SemaphoreType.DMA((18,)),
            pltpu.SemaphoreType.DMA((18,)),
        ],
        compiler_params=pltpu.CompilerParams(collective_id=0),
    )(x2, Wq, K_ext, V_ext, Wo)


# device time: 54785 ns/iter; 1.2595x vs baseline; 1.2595x over previous
import jax
import jax.numpy as jnp
from jax import lax
from jax.experimental import pallas as pl
from jax.experimental.pallas import tpu as pltpu

N_DEV = 8
SQ = 1024
DH = 128
HQ_PER = 8
D_MODEL = 1024
WINDOW = 128
KBAND = 3 * 128
SCALE = 0.08838834764831843
CHUNK = SQ // N_DEV

PART_ROWS = (384, 384, 256)
PART_OFF = (0, 384, 768)
PART_MASKS = ((4, 3, 1), (3, 1, 4), (1, 4, 3))
N_ROUNDS = 3
MAX_HALF = 192


def kernel(x, Wq, K_ext, V_ext, Wo):
    x2 = x[0]

    def body(x_ref, wq_ref, k_any, v_any, wo_ref, out_ref,
             acc_ref, ctx_ref, k_ref, v_ref, rs_buf, tx_buf, ag_buf,
             kv_sems, send_sems, recv_sems):
        my = lax.axis_index("i")

        def side(mask):
            if mask == 4:
                return (my // 4) % 2
            if mask == 3:
                return (my // 2) % 2
            return (my + my // 2) % 2

        kv_dmas = []
        for h in range(HQ_PER):
            hh = my * HQ_PER + h
            for src, dst, sem in ((k_any, k_ref, kv_sems.at[0, h]),
                                  (v_any, v_ref, kv_sems.at[1, h])):
                dma = pltpu.make_async_copy(src.at[0, :, hh, :],
                                            dst.at[h], sem)
                dma.start()
                kv_dmas.append(dma)

        barrier = pltpu.get_barrier_semaphore()
        for mask in (1, 3, 4):
            pl.semaphore_signal(barrier, inc=1, device_id=(my ^ mask,),
                                device_id_type=pl.DeviceIdType.MESH)
        pl.semaphore_wait(barrier, 3)

        q = jnp.dot(x_ref[...], wq_ref[...],
                    preferred_element_type=jnp.float32) * SCALE

        for dma in kv_dmas:
            dma.wait()

        masks = []
        for rel in (0, 128, 256):
            qi = rel + lax.broadcasted_iota(jnp.int32, (CHUNK, KBAND), 0)
            ki = lax.broadcasted_iota(jnp.int32, (CHUNK, KBAND), 1)
            masks.append(jnp.abs(qi - ki) <= WINDOW)
        for cc in range(N_DEV):
            row0 = cc * CHUNK
            start = min(max(row0 - WINDOW, 0), SQ - KBAND)
            band = masks[(row0 - start) // CHUNK]
            for h in range(HQ_PER):
                s = lax.dot_general(q[row0:row0 + CHUNK, h * DH:(h + 1) * DH],
                                    k_ref[h, start:start + KBAND, :],
                                    (((1,), (1,)), ((), ())),
                                    preferred_element_type=jnp.float32)
                w = jnp.exp(jnp.where(band, s, -1e9))
                ctx = jnp.dot(w, v_ref[h, start:start + KBAND, :],
                              preferred_element_type=jnp.float32)
                ctx_ref[pl.ds(row0, CHUNK), h * DH:(h + 1) * DH] = (
                    ctx / jnp.sum(w, axis=1, keepdims=True))

        acc_ref[...] = jnp.dot(ctx_ref[...], wo_ref[...],
                               preferred_element_type=jnp.float32)

        offs = [jnp.int32(0)] * 3
        sizes = list(PART_ROWS)
        pending = []
        for r in range(N_ROUNDS):
            started = []
            for p in range(3):
                mask = PART_MASKS[p][r]
                half = sizes[p] // 2
                b = side(mask)
                send_off = PART_OFF[p] + offs[p] + (1 - b) * half
                keep_off = PART_OFF[p] + offs[p] + b * half
                tx_buf[p, r, :half, :] = (
                    acc_ref[pl.ds(send_off, half), :].astype(jnp.bfloat16))
                rdma = pltpu.make_async_remote_copy(
                    src_ref=tx_buf.at[p, r, pl.ds(0, half), :],
                    dst_ref=rs_buf.at[p, r, pl.ds(0, half), :],
                    send_sem=send_sems.at[p * 6 + r],
                    recv_sem=recv_sems.at[p * 6 + r],
                    device_id=(my ^ mask,),
                    device_id_type=pl.DeviceIdType.MESH,
                )
                rdma.start()
                started.append((rdma, keep_off, half))
                offs[p] = offs[p] + b * half
                sizes[p] = half
            for p, (rdma, keep_off, half) in enumerate(started):
                rdma.wait_recv()
                sl = pl.ds(keep_off, half)
                acc_ref[sl, :] = (acc_ref[sl, :]
                                  + rs_buf[p, r, :half, :].astype(jnp.float32))
                pending.append(rdma)

        for p in range(3):
            sl = pl.ds(PART_OFF[p] + offs[p], sizes[p])
            ag_buf[sl, :] = acc_ref[sl, :].astype(jnp.bfloat16)

        for j in range(N_ROUNDS):
            started = []
            for p in range(3):
                mask = PART_MASKS[p][N_ROUNDS - 1 - j]
                b = side(mask)
                cur = sizes[p]
                sl = pl.ds(PART_OFF[p] + offs[p], cur)
                rdma = pltpu.make_async_remote_copy(
                    src_ref=ag_buf.at[sl, :],
                    dst_ref=ag_buf.at[sl, :],
                    send_sem=send_sems.at[p * 6 + N_ROUNDS + j],
                    recv_sem=recv_sems.at[p * 6 + N_ROUNDS + j],
                    device_id=(my ^ mask,),
                    device_id_type=pl.DeviceIdType.MESH,
                )
                rdma.start()
                started.append(rdma)
                offs[p] = offs[p] - b * cur
                sizes[p] = 2 * cur
            for rdma in started:
                rdma.wait_recv()
                pending.append(rdma)

        out_ref[0, :, :] = ag_buf[...].astype(jnp.float32)

        for rdma in pending:
            rdma.wait_send()

    return pl.pallas_call(
        body,
        out_shape=jax.ShapeDtypeStruct((1, SQ, D_MODEL), jnp.float32),
        in_specs=[
            pl.BlockSpec(memory_space=pltpu.VMEM),
            pl.BlockSpec(memory_space=pltpu.VMEM),
            pl.BlockSpec(memory_space=pltpu.MemorySpace.HBM),
            pl.BlockSpec(memory_space=pltpu.MemorySpace.HBM),
            pl.BlockSpec(memory_space=pltpu.VMEM),
        ],
        out_specs=pl.BlockSpec(memory_space=pltpu.VMEM),
        scratch_shapes=[
            pltpu.VMEM((SQ, D_MODEL), jnp.float32),
            pltpu.VMEM((SQ, D_MODEL), jnp.float32),
            pltpu.VMEM((HQ_PER, SQ, DH), jnp.float32),
            pltpu.VMEM((HQ_PER, SQ, DH), jnp.float32),
            pltpu.VMEM((3, N_ROUNDS, MAX_HALF, D_MODEL), jnp.bfloat16),
            pltpu.VMEM((3, N_ROUNDS, MAX_HALF, D_MODEL), jnp.bfloat16),
            pltpu.VMEM((SQ, D_MODEL), jnp.bfloat16),
            pltpu.SemaphoreType.DMA((2, HQ_PER)),
            pltpu.SemaphoreType.DMA((18,)),
            pltpu.SemaphoreType.DMA((18,)),
        ],
        compiler_params=pltpu.CompilerParams(collective_id=0),
    )(x2, Wq, K_ext, V_ext, Wo)
